# Initial kernel scaffold; baseline (speedup 1.0000x reference)
import functools

import jax
import jax.numpy as jnp
from jax import lax
from jax.experimental import pallas as pl
from jax.experimental.pallas import tpu as pltpu
from jax.experimental.pallas import tpu_sc as plsc

B, S, V, M, D = 4096, 200, 1000000, 200, 64
N = B * S
NC, NS, L = 2, 16, 16
NW = NC * NS
ROWS_PER_W = N // NW
CHUNK = 128
NCHUNK = ROWS_PER_W // CHUNK
DV = D // L


def _body(seq_hbm, pos_hbm, tok_hbm, ptab_hbm, w_hbm, b_hbm, out_hbm,
          idx_v, pidx_v, tok_buf, pos_buf, wv, bv, sem_t, sem_p):
    wid = lax.axis_index("s") * NC + lax.axis_index("c")
    base = wid * ROWS_PER_W

    pltpu.sync_copy(w_hbm, wv)
    pltpu.sync_copy(b_hbm, bv)
    w0 = [wv[pl.ds(j * L, L)] for j in range(DV)]
    w1 = [wv[pl.ds(D + j * L, L)] for j in range(DV)]
    bb = [bv[pl.ds(j * L, L)] for j in range(DV)]

    def chunk_body(g, _):
        row0 = base + g * CHUNK
        pltpu.sync_copy(seq_hbm.at[pl.ds(row0, CHUNK)], idx_v)
        pltpu.sync_copy(pos_hbm.at[pl.ds(row0, CHUNK)], pidx_v)
        cp_t = pltpu.async_copy(tok_hbm.at[idx_v], tok_buf, sem_t)
        cp_p = pltpu.async_copy(ptab_hbm.at[pidx_v], pos_buf, sem_p)
        cp_t.wait()
        cp_p.wait()

        def row_body(r, _):
            for j in range(DV):
                t = tok_buf[r, pl.ds(j * L, L)]
                p = pos_buf[r, pl.ds(j * L, L)]
                tok_buf[r, pl.ds(j * L, L)] = t * w0[j] + p * w1[j] + bb[j]
            return 0

        lax.fori_loop(0, CHUNK, row_body, 0)
        pltpu.sync_copy(tok_buf, out_hbm.at[pl.ds(row0, CHUNK)])
        return 0

    lax.fori_loop(0, NCHUNK, chunk_body, 0)


@jax.jit
def _run(seq_flat, pos_flat, token_table, pos_table, w_flat, bias):
    mesh = plsc.VectorSubcoreMesh(core_axis_name="c", subcore_axis_name="s")
    out = pl.kernel(
        _body,
        out_type=jax.ShapeDtypeStruct((N, D), jnp.float32),
        mesh=mesh,
        scratch_types=[
            pltpu.VMEM((CHUNK,), jnp.int32),
            pltpu.VMEM((CHUNK,), jnp.int32),
            pltpu.VMEM((CHUNK, D), jnp.float32),
            pltpu.VMEM((CHUNK, D), jnp.float32),
            pltpu.VMEM((2 * D,), jnp.float32),
            pltpu.VMEM((D,), jnp.float32),
            pltpu.SemaphoreType.DMA,
            pltpu.SemaphoreType.DMA,
        ],
    )(seq_flat, pos_flat, token_table, pos_table, w_flat, bias)
    return out


def kernel(sequence, position_ids, token_table, pos_table, embedding_weights,
           embedding_bias):
    seq_flat = sequence.reshape(N).astype(jnp.int32)
    pos_flat = position_ids.reshape(N).astype(jnp.int32)
    w_flat = embedding_weights.reshape(2 * D).astype(jnp.float32)
    out = _run(seq_flat, pos_flat, token_table, pos_table, w_flat,
               embedding_bias)
    return out.reshape(B, S, D)


# speedup vs baseline: 2.2546x; 2.2546x over previous
import functools

import jax
import jax.numpy as jnp
from jax import lax
from jax.experimental import pallas as pl
from jax.experimental.pallas import tpu as pltpu
from jax.experimental.pallas import tpu_sc as plsc

B, S, V, M, D = 4096, 200, 1000000, 200, 64
N = B * S
NC, NS, L = 2, 16, 16
NW = NC * NS
ROWS_PER_W = N // NW
CHUNK = 128
NCHUNK = ROWS_PER_W // CHUNK
DV = D // L


def _body(seq_hbm, pos_hbm, tok_hbm, ptab_hbm, w_hbm, b_hbm, out_hbm,
          idx_v, pidx_v, tok_buf, pos_buf, wv, bv, sem_t, sem_p):
    wid = lax.axis_index("s") * NC + lax.axis_index("c")
    base = wid * ROWS_PER_W

    pltpu.sync_copy(w_hbm, wv)
    pltpu.sync_copy(b_hbm, bv)
    w0 = [wv[pl.ds(j * L, L)] for j in range(DV)]
    w1 = [wv[pl.ds(D + j * L, L)] for j in range(DV)]
    bb = [bv[pl.ds(j * L, L)] for j in range(DV)]

    def chunk_body(g, _):
        row0 = base + g * CHUNK
        pltpu.sync_copy(seq_hbm.at[pl.ds(row0, CHUNK)], idx_v)
        pltpu.sync_copy(pos_hbm.at[pl.ds(row0, CHUNK)], pidx_v)
        cp_t = pltpu.async_copy(tok_hbm.at[idx_v], tok_buf, sem_t)
        cp_p = pltpu.async_copy(ptab_hbm.at[pidx_v], pos_buf, sem_p)
        cp_t.wait()
        cp_p.wait()

        def row_body(r, _):
            for j in range(DV):
                t = tok_buf[r, pl.ds(j * L, L)]
                p = pos_buf[r, pl.ds(j * L, L)]
                tok_buf[r, pl.ds(j * L, L)] = t * w0[j] + p * w1[j] + bb[j]
            return 0

        lax.fori_loop(0, CHUNK, row_body, 0)
        pltpu.sync_copy(tok_buf, out_hbm.at[pl.ds(row0, CHUNK)])
        return 0

    lax.fori_loop(0, NCHUNK, chunk_body, 0)


@jax.jit
def _run(seq_flat, pos_flat, token_table, pos_table, w_flat, bias):
    mesh = plsc.VectorSubcoreMesh(core_axis_name="c", subcore_axis_name="s")
    out = pl.kernel(
        _body,
        out_type=jax.ShapeDtypeStruct((N, D), jnp.float32),
        mesh=mesh,
        compiler_params=pltpu.CompilerParams(use_tc_tiling_on_sc=False),
        scratch_types=[
            pltpu.VMEM((CHUNK,), jnp.int32),
            pltpu.VMEM((CHUNK,), jnp.int32),
            pltpu.VMEM((CHUNK, D), jnp.float32),
            pltpu.VMEM((CHUNK, D), jnp.float32),
            pltpu.VMEM((2 * D,), jnp.float32),
            pltpu.VMEM((D,), jnp.float32),
            pltpu.SemaphoreType.DMA,
            pltpu.SemaphoreType.DMA,
        ],
    )(seq_flat, pos_flat, token_table, pos_table, w_flat, bias)
    return out


def kernel(sequence, position_ids, token_table, pos_table, embedding_weights,
           embedding_bias):
    seq_flat = sequence.reshape(N).astype(jnp.int32)
    pos_flat = position_ids.reshape(N).astype(jnp.int32)
    w_flat = embedding_weights.reshape(2 * D).astype(jnp.float32)
    out = _run(seq_flat, pos_flat, token_table, pos_table, w_flat,
               embedding_bias)
    return out.reshape(B, S, D)


# speedup vs baseline: 2.3639x; 1.0485x over previous
import jax
import jax.numpy as jnp
from jax import lax
from jax.experimental import pallas as pl
from jax.experimental.pallas import tpu as pltpu
from jax.experimental.pallas import tpu_sc as plsc

B, S, V, M, D = 4096, 200, 1000000, 200, 64
N = B * S
NC, NS, L = 2, 16, 16
NW = NC * NS
ROWS_PER_W = N // NW
CHUNK = 256
NCHUNK = ROWS_PER_W // CHUNK
NSEG = CHUNK // 128
DV = D // L


def _body(seq_hbm, pos_hbm, tok_hbm, ptab_hbm, w_hbm, b_hbm, out_hbm,
          idx0, idx1, pidx0, pidx1, tok0, tok1, pos0, pos1, ob0, ob1,
          wv, bv,
          sgt0, sgt1, sgp0, sgp1, ss0, ss1, si0, si1):
    idxs, pidxs = [idx0, idx1], [pidx0, pidx1]
    toks, poss, obs = [tok0, tok1], [pos0, pos1], [ob0, ob1]
    sgt, sgp, ss, si = [sgt0, sgt1], [sgp0, sgp1], [ss0, ss1], [si0, si1]

    wid = lax.axis_index("s") * NC + lax.axis_index("c")
    base = wid * ROWS_PER_W

    pltpu.sync_copy(w_hbm, wv)
    pltpu.sync_copy(b_hbm, bv)
    w0 = [wv[pl.ds(j * L, L)] for j in range(DV)]
    w1 = [wv[pl.ds(D + j * L, L)] for j in range(DV)]
    bb = [bv[pl.ds(j * L, L)] for j in range(DV)]

    def fire_gathers(b):
        for k in range(NSEG):
            sl = pl.ds(k * 128, 128)
            pltpu.async_copy(tok_hbm.at[idxs[b].at[sl]], toks[b].at[sl], sgt[b])
            pltpu.async_copy(ptab_hbm.at[pidxs[b].at[sl]], poss[b].at[sl], sgp[b])

    def wait_gathers(b):
        for k in range(NSEG):
            sl = pl.ds(k * 128, 128)
            pltpu.make_async_copy(tok_hbm.at[idxs[b].at[sl]], toks[b].at[sl],
                                  sgt[b]).wait()
            pltpu.make_async_copy(ptab_hbm.at[pidxs[b].at[sl]], poss[b].at[sl],
                                  sgp[b]).wait()

    def fire_idx(b, g):
        row0 = base + g * CHUNK
        pltpu.async_copy(seq_hbm.at[pl.ds(row0, CHUNK)], idxs[b], si[b])
        pltpu.async_copy(pos_hbm.at[pl.ds(row0, CHUNK)], pidxs[b], si[b])

    def wait_idx(b):
        pltpu.make_async_copy(seq_hbm.at[pl.ds(0, CHUNK)], idxs[b],
                              si[b]).wait()
        pltpu.make_async_copy(pos_hbm.at[pl.ds(0, CHUNK)], pidxs[b],
                              si[b]).wait()

    def fire_scatter(b, g):
        row0 = base + g * CHUNK
        pltpu.async_copy(obs[b], out_hbm.at[pl.ds(row0, CHUNK)], ss[b])

    def wait_scatter(b):
        pltpu.make_async_copy(obs[b], out_hbm.at[pl.ds(base, CHUNK)],
                              ss[b]).wait()

    def compute(b):
        def row_body(r, _):
            for j in range(DV):
                t = toks[b][r, pl.ds(j * L, L)]
                p = poss[b][r, pl.ds(j * L, L)]
                obs[b][r, pl.ds(j * L, L)] = t * w0[j] + p * w1[j] + bb[j]
            return 0
        lax.fori_loop(0, CHUNK, row_body, 0)

    for b in range(2):
        row0 = base + b * CHUNK
        pltpu.sync_copy(seq_hbm.at[pl.ds(row0, CHUNK)], idxs[b])
        pltpu.sync_copy(pos_hbm.at[pl.ds(row0, CHUNK)], pidxs[b])
        fire_gathers(b)

    def pair_body(gp, _):
        for b in range(2):
            g = gp * 2 + b
            wait_gathers(b)
            pref = g + 2 < NCHUNK

            @pl.when(pref)
            def _():
                fire_idx(b, g + 2)

            @pl.when(g >= 2)
            def _():
                wait_scatter(b)

            compute(b)
            fire_scatter(b, g)

            @pl.when(pref)
            def _():
                wait_idx(b)
                fire_gathers(b)
        return 0

    lax.fori_loop(0, NCHUNK // 2, pair_body, 0)
    for b in range(2):
        wait_scatter(b)


@jax.jit
def _run(seq_flat, pos_flat, token_table, pos_table, w_flat, bias):
    mesh = plsc.VectorSubcoreMesh(core_axis_name="c", subcore_axis_name="s")
    out = pl.kernel(
        _body,
        out_type=jax.ShapeDtypeStruct((N, D), jnp.float32),
        mesh=mesh,
        compiler_params=pltpu.CompilerParams(use_tc_tiling_on_sc=False),
        scratch_types=[
            pltpu.VMEM((CHUNK,), jnp.int32),
            pltpu.VMEM((CHUNK,), jnp.int32),
            pltpu.VMEM((CHUNK,), jnp.int32),
            pltpu.VMEM((CHUNK,), jnp.int32),
            pltpu.VMEM((CHUNK, D), jnp.float32),
            pltpu.VMEM((CHUNK, D), jnp.float32),
            pltpu.VMEM((CHUNK, D), jnp.float32),
            pltpu.VMEM((CHUNK, D), jnp.float32),
            pltpu.VMEM((CHUNK, D), jnp.float32),
            pltpu.VMEM((CHUNK, D), jnp.float32),
            pltpu.VMEM((2 * D,), jnp.float32),
            pltpu.VMEM((D,), jnp.float32),
            pltpu.SemaphoreType.DMA,
            pltpu.SemaphoreType.DMA,
            pltpu.SemaphoreType.DMA,
            pltpu.SemaphoreType.DMA,
            pltpu.SemaphoreType.DMA,
            pltpu.SemaphoreType.DMA,
            pltpu.SemaphoreType.DMA,
            pltpu.SemaphoreType.DMA,
        ],
    )(seq_flat, pos_flat, token_table, pos_table, w_flat, bias)
    return out


def kernel(sequence, position_ids, token_table, pos_table, embedding_weights,
           embedding_bias):
    seq_flat = sequence.reshape(N).astype(jnp.int32)
    pos_flat = position_ids.reshape(N).astype(jnp.int32)
    w_flat = embedding_weights.reshape(2 * D).astype(jnp.float32)
    out = _run(seq_flat, pos_flat, token_table, pos_table, w_flat,
               embedding_bias)
    return out.reshape(B, S, D)


# speedup vs baseline: 3.3699x; 1.4255x over previous
import jax
import jax.numpy as jnp
from jax import lax
from jax.experimental import pallas as pl
from jax.experimental.pallas import tpu as pltpu
from jax.experimental.pallas import tpu_sc as plsc

B, S, V, M, D = 4096, 200, 1000000, 200, 64
N = B * S
NC, NS, L = 2, 16, 16
NW = NC * NS
ROWS_PER_W = N // NW
CHUNK = 256
NCHUNK = ROWS_PER_W // CHUNK
NSEG = CHUNK // 128
DV = D // L


def _body(seq_hbm, pos_hbm, tok_hbm, ptab_hbm, w_hbm, b_hbm, out_hbm,
          idx0, idx1, pidx0, pidx1, tok0, tok1, pos0, pos1, ob0, ob1, posc, posc_sh,
          wv, bv,
          sgt0, sgt1, sgp0, sgp1, ss0, ss1, si0, si1):
    idxs, pidxs = [idx0, idx1], [pidx0, pidx1]
    toks, poss, obs = [tok0, tok1], [pos0, pos1], [ob0, ob1]
    sgt, sgp, ss, si = [sgt0, sgt1], [sgp0, sgp1], [ss0, ss1], [si0, si1]

    wid = lax.axis_index("s") * NC + lax.axis_index("c")
    base = wid * ROWS_PER_W

    pltpu.sync_copy(w_hbm, wv)
    pltpu.sync_copy(b_hbm, bv)
    w0 = [wv[pl.ds(j * L, L)] for j in range(DV)]
    w1 = [wv[pl.ds(D + j * L, L)] for j in range(DV)]
    bb = [bv[pl.ds(j * L, L)] for j in range(DV)]

    pltpu.sync_copy(ptab_hbm, posc)

    def posc_body(p, _):
        for j in range(DV):
            posc[p, pl.ds(j * L, L)] = posc[p, pl.ds(j * L, L)] * w1[j] + bb[j]
        return 0

    lax.fori_loop(0, M, posc_body, 0)

    @pl.when(lax.axis_index("s") == 0)
    def _():
        pltpu.sync_copy(posc, posc_sh)

    plsc.subcore_barrier()

    def fire_gathers(b):
        for k in range(NSEG):
            sl = pl.ds(k * 128, 128)
            pltpu.async_copy(tok_hbm.at[idxs[b].at[sl]], toks[b].at[sl], sgt[b])
            pltpu.async_copy(posc_sh.at[pidxs[b].at[sl]], poss[b].at[sl],
                             sgp[b])

    def wait_gathers(b):
        for k in range(NSEG):
            sl = pl.ds(k * 128, 128)
            pltpu.make_async_copy(tok_hbm.at[idxs[b].at[sl]], toks[b].at[sl],
                                  sgt[b]).wait()
            pltpu.make_async_copy(posc_sh.at[pidxs[b].at[sl]], poss[b].at[sl],
                                  sgp[b]).wait()

    def fire_idx(b, g):
        row0 = base + g * CHUNK
        pltpu.async_copy(seq_hbm.at[pl.ds(row0, CHUNK)], idxs[b], si[b])
        pltpu.async_copy(pos_hbm.at[pl.ds(row0, CHUNK)], pidxs[b], si[b])

    def wait_idx(b):
        pltpu.make_async_copy(seq_hbm.at[pl.ds(0, CHUNK)], idxs[b],
                              si[b]).wait()
        pltpu.make_async_copy(pos_hbm.at[pl.ds(0, CHUNK)], pidxs[b],
                              si[b]).wait()

    def fire_scatter(b, g):
        row0 = base + g * CHUNK
        pltpu.async_copy(obs[b], out_hbm.at[pl.ds(row0, CHUNK)], ss[b])

    def wait_scatter(b):
        pltpu.make_async_copy(obs[b], out_hbm.at[pl.ds(base, CHUNK)],
                              ss[b]).wait()

    def compute(b):
        def row_body(r, _):
            for j in range(DV):
                t = toks[b][r, pl.ds(j * L, L)]
                p = poss[b][r, pl.ds(j * L, L)]
                obs[b][r, pl.ds(j * L, L)] = t * w0[j] + p
            return 0
        lax.fori_loop(0, CHUNK, row_body, 0)

    for b in range(2):
        row0 = base + b * CHUNK
        pltpu.sync_copy(seq_hbm.at[pl.ds(row0, CHUNK)], idxs[b])
        pltpu.sync_copy(pos_hbm.at[pl.ds(row0, CHUNK)], pidxs[b])
        fire_gathers(b)

    def pair_body(gp, _):
        for b in range(2):
            g = gp * 2 + b
            wait_gathers(b)
            pref = g + 2 < NCHUNK

            @pl.when(pref)
            def _():
                fire_idx(b, g + 2)

            @pl.when(g >= 2)
            def _():
                wait_scatter(b)

            compute(b)
            fire_scatter(b, g)

            @pl.when(pref)
            def _():
                wait_idx(b)
                fire_gathers(b)
        return 0

    lax.fori_loop(0, NCHUNK // 2, pair_body, 0)
    for b in range(2):
        wait_scatter(b)


@jax.jit
def _run(seq_flat, pos_flat, token_table, pos_table, w_flat, bias):
    mesh = plsc.VectorSubcoreMesh(core_axis_name="c", subcore_axis_name="s")
    out = pl.kernel(
        _body,
        out_type=jax.ShapeDtypeStruct((N, D), jnp.float32),
        mesh=mesh,
        compiler_params=pltpu.CompilerParams(use_tc_tiling_on_sc=False),
        scratch_types=[
            pltpu.VMEM((CHUNK,), jnp.int32),
            pltpu.VMEM((CHUNK,), jnp.int32),
            pltpu.VMEM((CHUNK,), jnp.int32),
            pltpu.VMEM((CHUNK,), jnp.int32),
            pltpu.VMEM((CHUNK, D), jnp.float32),
            pltpu.VMEM((CHUNK, D), jnp.float32),
            pltpu.VMEM((CHUNK, D), jnp.float32),
            pltpu.VMEM((CHUNK, D), jnp.float32),
            pltpu.VMEM((CHUNK, D), jnp.float32),
            pltpu.VMEM((CHUNK, D), jnp.float32),
            pltpu.VMEM((M, D), jnp.float32),
            pltpu.VMEM_SHARED((M, D), jnp.float32),
            pltpu.VMEM((2 * D,), jnp.float32),
            pltpu.VMEM((D,), jnp.float32),
            pltpu.SemaphoreType.DMA,
            pltpu.SemaphoreType.DMA,
            pltpu.SemaphoreType.DMA,
            pltpu.SemaphoreType.DMA,
            pltpu.SemaphoreType.DMA,
            pltpu.SemaphoreType.DMA,
            pltpu.SemaphoreType.DMA,
            pltpu.SemaphoreType.DMA,
        ],
    )(seq_flat, pos_flat, token_table, pos_table, w_flat, bias)
    return out


def kernel(sequence, position_ids, token_table, pos_table, embedding_weights,
           embedding_bias):
    seq_flat = sequence.reshape(N).astype(jnp.int32)
    pos_flat = position_ids.reshape(N).astype(jnp.int32)
    w_flat = embedding_weights.reshape(2 * D).astype(jnp.float32)
    out = _run(seq_flat, pos_flat, token_table, pos_table, w_flat,
               embedding_bias)
    return out.reshape(B, S, D)
